# baseline (speedup 1.0000x reference)
import functools

import jax
import jax.numpy as jnp
from jax import lax
from jax.experimental import pallas as pl
from jax.experimental.pallas import tpu as pltpu
from jax.experimental.pallas import tpu_sc as plsc

_BATCH = 16384
_DIM = 64
_NREL = 1000
_NC = 2
_NS = 16
_L = 16
_NW = _NC * _NS
_BPW = _BATCH // _NW


_HALF = _BPW // 2


def _sc_body(head_hbm, idx_hbm, tail_hbm, table_hbm, out_hbm,
             table_v, head_v, tail_v, idx_v, out_v, sem):
    wid = lax.axis_index("s") * _NC + lax.axis_index("c")
    base = wid * _BPW

    pltpu.async_copy(table_hbm, table_v, sem).wait()
    pltpu.sync_copy(idx_hbm.at[pl.ds(base, _BPW)], idx_v)

    lane = lax.iota(jnp.int32, _L)

    for half in range(2):
        hb = (base + half * _HALF) * _DIM
        copies = [
            pltpu.async_copy(head_hbm.at[pl.ds(hb, _HALF * _DIM)], head_v, sem),
            pltpu.async_copy(tail_hbm.at[pl.ds(hb, _HALF * _DIM)], tail_v, sem),
        ]
        for cp in copies:
            cp.wait()

        def group(g, _, half=half):
            b0 = g * _L
            row_a = (b0 * _DIM) + lane * _DIM
            rel_a = idx_v[pl.ds(half * _HALF + b0, _L)] * _DIM
            accs = [jnp.zeros((_L,), jnp.float32) for _ in range(4)]
            for d in range(_DIM):
                h = plsc.load_gather(head_v, [row_a + d])
                r = plsc.load_gather(table_v, [rel_a + d])
                t = plsc.load_gather(tail_v, [row_a + d])
                accs[d % 4] = accs[d % 4] + h * r * t
            acc = (accs[0] + accs[1]) + (accs[2] + accs[3])
            out_v[pl.ds(half * _HALF + b0, _L)] = 1.0 / (1.0 + jnp.exp(-acc))
            return 0

        lax.fori_loop(0, _HALF // _L, group, 0)

    pltpu.sync_copy(out_v, out_hbm.at[pl.ds(base, _BPW)])


@jax.jit
def _dist_mult_sc(head_flat, rel_idx, tail_flat, table_flat):
    mesh = plsc.VectorSubcoreMesh(core_axis_name="c", subcore_axis_name="s")
    run = functools.partial(
        pl.kernel,
        out_type=jax.ShapeDtypeStruct((_BATCH,), jnp.float32),
        mesh=mesh,
        compiler_params=pltpu.CompilerParams(needs_layout_passes=False),
        scratch_types=[
            pltpu.VMEM((_NREL * _DIM,), jnp.float32),
            pltpu.VMEM((_HALF * _DIM,), jnp.float32),
            pltpu.VMEM((_HALF * _DIM,), jnp.float32),
            pltpu.VMEM((_BPW,), jnp.int32),
            pltpu.VMEM((_BPW,), jnp.float32),
            pltpu.SemaphoreType.DMA,
        ],
    )(_sc_body)
    return run(head_flat, rel_idx, tail_flat, table_flat)


def kernel(head_e, rel_idx, tail_e, kernel):
    score = _dist_mult_sc(head_e.reshape(-1), rel_idx.astype(jnp.int32),
                          tail_e.reshape(-1), kernel.reshape(-1))
    return score.reshape(1, _BATCH)


# speedup vs baseline: 1.5251x; 1.5251x over previous
import functools

import jax
import jax.numpy as jnp
from jax import lax
from jax.experimental import pallas as pl
from jax.experimental.pallas import tpu as pltpu
from jax.experimental.pallas import tpu_sc as plsc

_BATCH = 16384
_DIM = 64
_NREL = 1000
_NC = 2
_NS = 16
_L = 16
_NW = _NC * _NS
_BPW = _BATCH // _NW


_HALF = _BPW // 2


def _sc_body(head_hbm, idx_hbm, tail_hbm, table_hbm, out_hbm,
             table_v, head_v, tail_v, idx_v, out_v, sem):
    wid = lax.axis_index("s") * _NC + lax.axis_index("c")
    base = wid * _BPW

    pltpu.async_copy(table_hbm, table_v, sem).wait()
    pltpu.sync_copy(idx_hbm.at[pl.ds(base, _BPW)], idx_v)

    lane = lax.iota(jnp.int32, _L)

    for half in range(2):
        hb = (base + half * _HALF) * _DIM
        copies = [
            pltpu.async_copy(head_hbm.at[pl.ds(hb, _HALF * _DIM)], head_v, sem),
            pltpu.async_copy(tail_hbm.at[pl.ds(hb, _HALF * _DIM)], tail_v, sem),
        ]
        for cp in copies:
            cp.wait()

        def group(g, _, half=half):
            b0 = g * _L
            row_a = (b0 * _DIM) + lane * _DIM
            rel_a = idx_v[pl.ds(half * _HALF + b0, _L)] * _DIM
            accs = [jnp.zeros((_L,), jnp.float32) for _ in range(4)]
            for d in range(_DIM):
                dvec = (lane + d) & (_DIM - 1)
                h = plsc.load_gather(head_v, [row_a + dvec])
                r = plsc.load_gather(table_v, [rel_a + dvec])
                t = plsc.load_gather(tail_v, [row_a + dvec])
                accs[d % 4] = accs[d % 4] + h * r * t
            acc = (accs[0] + accs[1]) + (accs[2] + accs[3])
            out_v[pl.ds(half * _HALF + b0, _L)] = 1.0 / (1.0 + jnp.exp(-acc))
            return 0

        lax.fori_loop(0, _HALF // _L, group, 0)

    pltpu.sync_copy(out_v, out_hbm.at[pl.ds(base, _BPW)])


@jax.jit
def _dist_mult_sc(head_flat, rel_idx, tail_flat, table_flat):
    mesh = plsc.VectorSubcoreMesh(core_axis_name="c", subcore_axis_name="s")
    run = functools.partial(
        pl.kernel,
        out_type=jax.ShapeDtypeStruct((_BATCH,), jnp.float32),
        mesh=mesh,
        compiler_params=pltpu.CompilerParams(needs_layout_passes=False),
        scratch_types=[
            pltpu.VMEM((_NREL * _DIM,), jnp.float32),
            pltpu.VMEM((_HALF * _DIM,), jnp.float32),
            pltpu.VMEM((_HALF * _DIM,), jnp.float32),
            pltpu.VMEM((_BPW,), jnp.int32),
            pltpu.VMEM((_BPW,), jnp.float32),
            pltpu.SemaphoreType.DMA,
        ],
    )(_sc_body)
    return run(head_flat, rel_idx, tail_flat, table_flat)


def kernel(head_e, rel_idx, tail_e, kernel):
    score = _dist_mult_sc(head_e.reshape(-1), rel_idx.astype(jnp.int32),
                          tail_e.reshape(-1), kernel.reshape(-1))
    return score.reshape(1, _BATCH)


# speedup vs baseline: 1.8590x; 1.2189x over previous
import functools

import jax
import jax.numpy as jnp
from jax import lax
from jax.experimental import pallas as pl
from jax.experimental.pallas import tpu as pltpu
from jax.experimental.pallas import tpu_sc as plsc

_BATCH = 16384
_DIM = 64
_NREL = 1000
_NC = 2
_NS = 16
_L = 16
_NW = _NC * _NS
_BPW = _BATCH // _NW
_CH = 128
_NCHUNK = _BPW // _CH


def _sc_body(head_hbm, idx_hbm, tail_hbm, table_hbm, out_hbm,
             idx_v, out_v, h0, h1, t0, t1, r0, r1, sem0, sem1):
    wid = lax.axis_index("s") * _NC + lax.axis_index("c")
    base = wid * _BPW

    pltpu.sync_copy(idx_hbm.at[pl.ds(base, _BPW)], idx_v)

    hbufs, tbufs, rbufs = (h0, h1), (t0, t1), (r0, r1)
    sems = (sem0, sem1)
    lane = lax.iota(jnp.int32, _L)

    def fire(c):
        slot = c % 2
        rb = base + c * _CH
        return [
            pltpu.async_copy(table_hbm.at[idx_v.at[pl.ds(c * _CH, _CH)]],
                             rbufs[slot], sems[slot]),
            pltpu.async_copy(head_hbm.at[pl.ds(rb, _CH)], hbufs[slot], sems[slot]),
            pltpu.async_copy(tail_hbm.at[pl.ds(rb, _CH)], tbufs[slot], sems[slot]),
        ]

    inflight = {0: fire(0)}
    for c in range(_NCHUNK):
        slot = c % 2
        if c + 1 < _NCHUNK:
            inflight[c + 1] = fire(c + 1)
        for cp in inflight.pop(c):
            cp.wait()

        hv, tv, rv = hbufs[slot], tbufs[slot], rbufs[slot]

        def group(g, _, c=c, hv=hv, tv=tv, rv=rv):
            rows = g * _L + lane
            accs = [jnp.zeros((_L,), jnp.float32) for _ in range(4)]
            for d in range(_DIM):
                dvec = (lane + d) & (_DIM - 1)
                h = plsc.load_gather(hv, [rows, dvec])
                r = plsc.load_gather(rv, [rows, dvec])
                t = plsc.load_gather(tv, [rows, dvec])
                accs[d % 4] = accs[d % 4] + h * r * t
            acc = (accs[0] + accs[1]) + (accs[2] + accs[3])
            out_v[pl.ds(c * _CH + g * _L, _L)] = 1.0 / (1.0 + jnp.exp(-acc))
            return 0

        lax.fori_loop(0, _CH // _L, group, 0)

    pltpu.sync_copy(out_v, out_hbm.at[pl.ds(base, _BPW)])


@jax.jit
def _dist_mult_sc(head_e, rel_idx, tail_e, table):
    mesh = plsc.VectorSubcoreMesh(core_axis_name="c", subcore_axis_name="s")
    run = functools.partial(
        pl.kernel,
        out_type=jax.ShapeDtypeStruct((_BATCH,), jnp.float32),
        mesh=mesh,
        compiler_params=pltpu.CompilerParams(needs_layout_passes=False),
        scratch_types=[
            pltpu.VMEM((_BPW,), jnp.int32),
            pltpu.VMEM((_BPW,), jnp.float32),
            pltpu.VMEM((_CH, _DIM), jnp.float32),
            pltpu.VMEM((_CH, _DIM), jnp.float32),
            pltpu.VMEM((_CH, _DIM), jnp.float32),
            pltpu.VMEM((_CH, _DIM), jnp.float32),
            pltpu.VMEM((_CH, 2 * _DIM), jnp.float32),
            pltpu.VMEM((_CH, 2 * _DIM), jnp.float32),
            pltpu.SemaphoreType.DMA,
            pltpu.SemaphoreType.DMA,
        ],
    )(_sc_body)
    return run(head_e, rel_idx, tail_e, table)


def kernel(head_e, rel_idx, tail_e, kernel):
    table_p = jnp.pad(kernel, ((0, 0), (0, _DIM)))
    score = _dist_mult_sc(head_e, rel_idx.astype(jnp.int32), tail_e, table_p)
    return score.reshape(1, _BATCH)


# speedup vs baseline: 1.8634x; 1.0024x over previous
import functools

import jax
import jax.numpy as jnp
from jax import lax
from jax.experimental import pallas as pl
from jax.experimental.pallas import tpu as pltpu
from jax.experimental.pallas import tpu_sc as plsc

_BATCH = 16384
_DIM = 64
_NREL = 1000
_NC = 2
_NS = 16
_L = 16
_NW = _NC * _NS
_BPW = _BATCH // _NW
_CH = 128
_NCHUNK = _BPW // _CH


def _sc_body(head_hbm, idx_hbm, tail_hbm, table_hbm, out_hbm,
             idx_v, out_v, h0, h1, t0, t1, r0, r1, sem0, sem1):
    wid = lax.axis_index("s") * _NC + lax.axis_index("c")
    base = wid * _BPW

    pltpu.sync_copy(idx_hbm.at[pl.ds(base, _BPW)], idx_v)

    hbufs, tbufs, rbufs = (h0, h1), (t0, t1), (r0, r1)
    sems = (sem0, sem1)
    lane = lax.iota(jnp.int32, _L)

    def fire(c):
        slot = c % 2
        rb = base + c * _CH
        return [
            pltpu.async_copy(table_hbm.at[idx_v.at[pl.ds(c * _CH, _CH)]],
                             rbufs[slot], sems[slot]),
            pltpu.async_copy(head_hbm.at[pl.ds(rb, _CH)], hbufs[slot], sems[slot]),
            pltpu.async_copy(tail_hbm.at[pl.ds(rb, _CH)], tbufs[slot], sems[slot]),
        ]

    inflight = {0: fire(0)}
    for c in range(_NCHUNK):
        slot = c % 2
        if c + 1 < _NCHUNK:
            inflight[c + 1] = fire(c + 1)
        for cp in inflight.pop(c):
            cp.wait()

        hv, tv, rv = hbufs[slot], tbufs[slot], rbufs[slot]

        def group(g, _, c=c, hv=hv, tv=tv, rv=rv):
            rows = g * _L + lane
            accs = [jnp.zeros((_L,), jnp.float32) for _ in range(4)]
            for d in range(_DIM):
                dvec = (lane + d) & (_DIM - 1)
                h = plsc.load_gather(hv, [rows, dvec])
                r = plsc.load_gather(rv, [rows, dvec])
                t = plsc.load_gather(tv, [rows, dvec])
                accs[d % 4] = accs[d % 4] + h * r * t
            acc = (accs[0] + accs[1]) + (accs[2] + accs[3])
            out_v[pl.ds(c * _CH + g * _L, _L)] = 1.0 / (1.0 + jnp.exp(-acc))
            return 0

        lax.fori_loop(0, _CH // _L, group, 0)

    pltpu.sync_copy(out_v, out_hbm.at[pl.ds(base, _BPW)])


@jax.jit
def _dist_mult_sc(head_e, rel_idx, tail_e, table):
    mesh = plsc.VectorSubcoreMesh(core_axis_name="c", subcore_axis_name="s")
    run = functools.partial(
        pl.kernel,
        out_type=jax.ShapeDtypeStruct((_BATCH,), jnp.float32),
        mesh=mesh,
        compiler_params=pltpu.CompilerParams(needs_layout_passes=False,
                                             use_tc_tiling_on_sc=True),
        scratch_types=[
            pltpu.VMEM((_BPW,), jnp.int32),
            pltpu.VMEM((_BPW,), jnp.float32),
            pltpu.VMEM((_CH, _DIM), jnp.float32),
            pltpu.VMEM((_CH, _DIM), jnp.float32),
            pltpu.VMEM((_CH, _DIM), jnp.float32),
            pltpu.VMEM((_CH, _DIM), jnp.float32),
            pltpu.VMEM((_CH, 2 * _DIM), jnp.float32),
            pltpu.VMEM((_CH, 2 * _DIM), jnp.float32),
            pltpu.SemaphoreType.DMA,
            pltpu.SemaphoreType.DMA,
        ],
    )(_sc_body)
    return run(head_e, rel_idx, tail_e, table)


def kernel(head_e, rel_idx, tail_e, kernel):
    table_p = jnp.pad(kernel, ((0, 0), (0, _DIM)))
    score = _dist_mult_sc(head_e, rel_idx.astype(jnp.int32), tail_e, table_p)
    return score.reshape(1, _BATCH)
